# baseline (speedup 1.0000x reference)
import functools

import jax
import jax.numpy as jnp
from jax import lax
from jax.experimental import pallas as pl
from jax.experimental.pallas import tpu as pltpu
from jax.experimental.pallas import tpu_sc as plsc

N = 10000
D = 128
E = 320000
OUT = 2

NC = 2
NS = 16
CN = 640
NP = NS * CN
EW1 = E // NS
EW3 = E // (NC * NS)
CH = 5
CE = EW3 // CH
L = 16


def _sc_coeffs(edge_flat):
  mesh = plsc.VectorSubcoreMesh(
      core_axis_name="c", subcore_axis_name="s", num_cores=NC)

  @functools.partial(
      pl.kernel,
      out_type=jax.ShapeDtypeStruct((3 * NP,), jnp.float32),
      mesh=mesh,
      scratch_types=[
          pltpu.VMEM((EW1,), jnp.int32),
          pltpu.VMEM((EW1,), jnp.float32),
          pltpu.VMEM((CN,), jnp.float32),
          pltpu.VMEM((CN,), jnp.float32),
          *[pltpu.VMEM((CE,), jnp.int32) for _ in range(CH)],
          *[pltpu.VMEM((CE,), jnp.int32) for _ in range(CH)],
          *[pltpu.VMEM((CE,), jnp.float32) for _ in range(CH)],
          pltpu.VMEM_SHARED((NP,), jnp.float32),
          pltpu.VMEM_SHARED((NP,), jnp.float32),
          pltpu.VMEM_SHARED((NP,), jnp.float32),
          pltpu.SemaphoreType.DMA,
          pltpu.SemaphoreType.DMA,
          pltpu.SemaphoreType.DMA,
          pltpu.SemaphoreType.DMA,
          pltpu.SemaphoreType.DMA,
      ],
  )
  def k(edge_hbm, out_hbm, dst1_v, ones_v, loc_v, dinv_v, *rest):
    dst3_v = rest[0:CH]
    src3_v = rest[CH:2 * CH]
    val_v = rest[2 * CH:3 * CH]
    deg_sh, s_sh, dinv_sh, sem0, sem1, sem2, semg, semsc = rest[3 * CH:]
    cid = lax.axis_index("c")
    wid = lax.axis_index("s")
    base_n = wid * CN
    base1 = wid * EW1
    base3 = (cid * NS + wid) * EW3

    cp1 = pltpu.async_copy(edge_hbm.at[pl.ds(E + base1, EW1)], dst1_v, sem0)
    cp3d = [pltpu.async_copy(edge_hbm.at[pl.ds(E + base3 + c * CE, CE)],
                             dst3_v[c], sem1) for c in range(CH)]
    cp3s = [pltpu.async_copy(edge_hbm.at[pl.ds(base3 + c * CE, CE)],
                             src3_v[c], sem2) for c in range(CH)]

    def fill_ones(i, carry):
      ones_v[pl.ds(i * L, L)] = jnp.full((L,), 1.0, jnp.float32)
      return carry
    lax.fori_loop(0, EW1 // L, fill_ones, 0, unroll=8)
    for j in range(CN // L):
      loc_v[pl.ds(j * L, L)] = jnp.zeros((L,), jnp.float32)

    pltpu.sync_copy(loc_v, deg_sh.at[pl.ds(base_n, CN)])
    pltpu.sync_copy(loc_v, s_sh.at[pl.ds(base_n, CN)])
    plsc.subcore_barrier()

    cp1.wait()
    pltpu.sync_copy(ones_v, deg_sh.at[dst1_v], add=True)
    plsc.subcore_barrier()

    pltpu.sync_copy(deg_sh.at[pl.ds(base_n, CN)], loc_v)
    for j in range(CN // L):
      dg = loc_v[pl.ds(j * L, L)] + 1.0
      y = jnp.full((L,), 0.5, jnp.float32)
      for kk in range(1, 10):
        y = jnp.where(dg >= float(4.0 ** kk), float(0.5 ** (kk + 1)), y)
      for _ in range(6):
        y = y * (1.5 - 0.5 * dg * y * y)
      dinv_v[pl.ds(j * L, L)] = y
    pltpu.sync_copy(dinv_v, dinv_sh.at[pl.ds(base_n, CN)])
    plsc.subcore_barrier()

    for c in range(CH):
      cp3d[c].wait()
      cp3s[c].wait()
    gd = [pltpu.async_copy(dinv_sh.at[dst3_v[c]], val_v[c], semg)
          for c in range(CH)]
    sc = []
    for c in range(CH):
      gd[c].wait()
      sc.append(pltpu.async_copy(val_v[c], s_sh.at[src3_v[c]], semsc,
                                 add=True))
    for c in range(CH):
      sc[c].wait()
    plsc.subcore_barrier()

    pltpu.sync_copy(s_sh.at[pl.ds(base_n, CN)],
                    out_hbm.at[pl.ds(cid * NP + base_n, CN)])
    @pl.when(cid == 0)
    def _():
      pltpu.sync_copy(dinv_v, out_hbm.at[pl.ds(2 * NP + base_n, CN)])

  return k(edge_flat)


def _tc_body(x_ref, p_ref, w0_ref, wih_ref, whh_ref, bih_ref, bhh_ref,
             lw_ref, lb_ref, out_ref):
  w0 = w0_ref[...]
  gates = lax.dot_general(
      w0, wih_ref[...] + whh_ref[...], (((1,), (1,)), ((), ())),
      preferred_element_type=jnp.float32)
  gates = gates + bih_ref[...] + bhh_ref[...]
  ig = jax.nn.sigmoid(gates[:, 0:D])
  fg = jax.nn.sigmoid(gates[:, D:2 * D])
  gg = jnp.tanh(gates[:, 2 * D:3 * D])
  og = jax.nn.sigmoid(gates[:, 3 * D:4 * D])
  cell = fg * w0 + ig * gg
  w_ev = og * jnp.tanh(cell)
  sv = p_ref[0:1, :] + p_ref[1:2, :]
  dv = p_ref[2:3, :]
  c = dv * (sv + dv)
  v = lax.dot_general(c[:, 0:N], x_ref[...], (((1,), (0,)), ((), ())),
                      preferred_element_type=jnp.float32)
  g = lax.dot_general(v, w_ev, (((1,), (0,)), ((), ())),
                      preferred_element_type=jnp.float32) * (1.0 / N)
  out_ref[...] = lax.dot_general(g, lw_ref[...], (((1,), (1,)), ((), ())),
                                 preferred_element_type=jnp.float32) + lb_ref[...]


def kernel(x, edge_index, initial_weight, lstm_W_ih, lstm_W_hh,
           lstm_b_ih, lstm_b_hh, lin_W, lin_b):
  packed = _sc_coeffs(edge_index.reshape(2 * E)).reshape(3, NP)
  return pl.pallas_call(
      _tc_body,
      out_shape=jax.ShapeDtypeStruct((1, OUT), jnp.float32),
  )(x, packed, initial_weight, lstm_W_ih, lstm_W_hh,
    lstm_b_ih.reshape(1, 4 * D), lstm_b_hh.reshape(1, 4 * D),
    lin_W, lin_b.reshape(1, OUT))


# speedup vs baseline: 1.0100x; 1.0100x over previous
import functools

import jax
import jax.numpy as jnp
from jax import lax
from jax.experimental import pallas as pl
from jax.experimental.pallas import tpu as pltpu
from jax.experimental.pallas import tpu_sc as plsc

N = 10000
D = 128
E = 320000
OUT = 2

NC = 2
NS = 16
CN = 640
NP = NS * CN
EW1 = E // NS
EW3 = E // (NC * NS)
L = 16


def _sc_coeffs(edge_flat):
  mesh = plsc.VectorSubcoreMesh(
      core_axis_name="c", subcore_axis_name="s", num_cores=NC)

  @functools.partial(
      pl.kernel,
      out_type=jax.ShapeDtypeStruct((3 * NP,), jnp.float32),
      mesh=mesh,
      scratch_types=[
          pltpu.VMEM((EW1,), jnp.int32),
          pltpu.VMEM((EW3,), jnp.int32),
          pltpu.VMEM((EW3,), jnp.int32),
          pltpu.VMEM((EW1,), jnp.float32),
          pltpu.VMEM((EW3,), jnp.float32),
          pltpu.VMEM((CN,), jnp.float32),
          pltpu.VMEM((CN,), jnp.float32),
          pltpu.VMEM_SHARED((NP,), jnp.float32),
          pltpu.VMEM_SHARED((NP,), jnp.float32),
          pltpu.VMEM_SHARED((NP,), jnp.float32),
          pltpu.SemaphoreType.DMA,
          pltpu.SemaphoreType.DMA,
          pltpu.SemaphoreType.DMA,
      ],
  )
  def k(edge_hbm, out_hbm, dst1_v, dst3_v, src3_v, ones_v, val_v, loc_v,
        dinv_v, deg_sh, s_sh, dinv_sh, sem0, sem1, sem2):
    cid = lax.axis_index("c")
    wid = lax.axis_index("s")
    base_n = wid * CN
    base1 = wid * EW1
    base3 = (cid * NS + wid) * EW3

    cp1 = pltpu.async_copy(edge_hbm.at[pl.ds(E + base1, EW1)], dst1_v, sem0)
    cp3d = pltpu.async_copy(edge_hbm.at[pl.ds(E + base3, EW3)], dst3_v, sem1)
    cp3s = pltpu.async_copy(edge_hbm.at[pl.ds(base3, EW3)], src3_v, sem2)

    def fill_ones(i, carry):
      ones_v[pl.ds(i * L, L)] = jnp.full((L,), 1.0, jnp.float32)
      return carry
    lax.fori_loop(0, EW1 // L, fill_ones, 0, unroll=8)
    for j in range(CN // L):
      loc_v[pl.ds(j * L, L)] = jnp.zeros((L,), jnp.float32)

    pltpu.sync_copy(loc_v, deg_sh.at[pl.ds(base_n, CN)])
    pltpu.sync_copy(loc_v, s_sh.at[pl.ds(base_n, CN)])
    plsc.subcore_barrier()

    cp1.wait()
    pltpu.sync_copy(ones_v, deg_sh.at[dst1_v], add=True)
    plsc.subcore_barrier()

    pltpu.sync_copy(deg_sh.at[pl.ds(base_n, CN)], loc_v)
    for j in range(CN // L):
      dg = loc_v[pl.ds(j * L, L)] + 1.0
      y = jnp.full((L,), 0.5, jnp.float32)
      for kk in range(1, 10):
        y = jnp.where(dg >= float(4.0 ** kk), float(0.5 ** (kk + 1)), y)
      for _ in range(6):
        y = y * (1.5 - 0.5 * dg * y * y)
      dinv_v[pl.ds(j * L, L)] = y
    pltpu.sync_copy(dinv_v, dinv_sh.at[pl.ds(base_n, CN)])
    plsc.subcore_barrier()

    cp3d.wait()
    pltpu.sync_copy(dinv_sh.at[dst3_v], val_v)
    cp3s.wait()
    pltpu.sync_copy(val_v, s_sh.at[src3_v], add=True)
    plsc.subcore_barrier()

    pltpu.sync_copy(s_sh.at[pl.ds(base_n, CN)],
                    out_hbm.at[pl.ds(cid * NP + base_n, CN)])
    @pl.when(cid == 0)
    def _():
      pltpu.sync_copy(dinv_v, out_hbm.at[pl.ds(2 * NP + base_n, CN)])

  return k(edge_flat)


def _tc_body(x_ref, p_ref, w0_ref, wih_ref, whh_ref, bih_ref, bhh_ref,
             lw_ref, lb_ref, out_ref):
  w0 = w0_ref[...]
  gates = lax.dot_general(
      w0, wih_ref[...] + whh_ref[...], (((1,), (1,)), ((), ())),
      preferred_element_type=jnp.float32)
  gates = gates + bih_ref[...] + bhh_ref[...]
  ig = jax.nn.sigmoid(gates[:, 0:D])
  fg = jax.nn.sigmoid(gates[:, D:2 * D])
  gg = jnp.tanh(gates[:, 2 * D:3 * D])
  og = jax.nn.sigmoid(gates[:, 3 * D:4 * D])
  cell = fg * w0 + ig * gg
  w_ev = og * jnp.tanh(cell)
  sv = p_ref[0:1, :] + p_ref[1:2, :]
  dv = p_ref[2:3, :]
  c = dv * (sv + dv)
  v = lax.dot_general(c[:, 0:N], x_ref[...], (((1,), (0,)), ((), ())),
                      preferred_element_type=jnp.float32)
  g = lax.dot_general(v, w_ev, (((1,), (0,)), ((), ())),
                      preferred_element_type=jnp.float32) * (1.0 / N)
  out_ref[...] = lax.dot_general(g, lw_ref[...], (((1,), (1,)), ((), ())),
                                 preferred_element_type=jnp.float32) + lb_ref[...]


def kernel(x, edge_index, initial_weight, lstm_W_ih, lstm_W_hh,
           lstm_b_ih, lstm_b_hh, lin_W, lin_b):
  packed = _sc_coeffs(edge_index.reshape(2 * E)).reshape(3, NP)
  return pl.pallas_call(
      _tc_body,
      out_shape=jax.ShapeDtypeStruct((1, OUT), jnp.float32),
  )(x, packed, initial_weight, lstm_W_ih, lstm_W_hh,
    lstm_b_ih.reshape(1, 4 * D), lstm_b_hh.reshape(1, 4 * D),
    lin_W, lin_b.reshape(1, OUT))
